# Initial kernel scaffold; baseline (speedup 1.0000x reference)
import jax
import jax.numpy as jnp
from jax.experimental import pallas as pl


def kernel(data, batch_id, depth):
    raise NotImplementedError("write your pallas kernel here")


# speedup vs baseline: 9.2676x; 9.2676x over previous
import jax
import jax.numpy as jnp
from jax import lax
from jax.experimental import pallas as pl
from jax.experimental.pallas import tpu as pltpu
from jax.experimental.pallas import tpu_sc as plsc

NSEG = 32
N = 320000
D = 128
L = 16
NC, NS = 2, 16
NW = NC * NS
ROWS_W = N // NW
CHUNK = 400
NCHUNK = ROWS_W // CHUNK
DSL = D // L

_mesh = plsc.VectorSubcoreMesh(
    core_axis_name="c", subcore_axis_name="s", num_cores=NC, num_subcores=NS
)


def _sc_body(data_hbm, bid_hbm, sums_hbm, cnts_hbm, data_v, bid_v, acc_tab, cnt_tab):
    wid = lax.axis_index("s") * NC + lax.axis_index("c")
    w0 = wid * ROWS_W
    lanes = lax.iota(jnp.int32, L)
    zf = jnp.zeros((L,), jnp.float32)
    lane0 = lanes == 0

    def _zero(i, _):
        acc_tab[pl.ds(i * L, L)] = zf
        return 0

    lax.fori_loop(0, NSEG * D // L, _zero, 0)
    cnt_tab[pl.ds(0, L)] = zf
    cnt_tab[pl.ds(L, L)] = zf

    pltpu.sync_copy(bid_hbm.at[pl.ds(w0, ROWS_W)], bid_v)


    def _flush(cur, cnt, acc):
        for j in range(DSL):
            idx = cur * D + j * L + lanes
            plsc.addupdate_scatter(acc_tab, [idx], acc[j])
        cidx = jnp.zeros((L,), jnp.int32) + cur
        plsc.addupdate_scatter(cnt_tab, [cidx], zf + cnt, mask=lane0)

    zeros_acc = tuple(jnp.zeros((L,), jnp.float32) for _ in range(DSL))
    zero_cnt = jnp.zeros((), jnp.float32)

    def _chunk_body(c, carry):
        cur, cnt, acc = carry
        pltpu.sync_copy(
            data_hbm.at[pl.ds((w0 + c * CHUNK) * D, CHUNK * D)], data_v
        )
        first = bid_v[pl.ds(c * CHUNK, L)][0]
        last = bid_v[pl.ds(c * CHUNK + CHUNK - L, L)][L - 1]
        uniform = first == last
        cont = jnp.logical_and(uniform, first == cur)

        @pl.when(jnp.logical_not(cont))
        def _():
            _flush(cur, cnt, acc)

        keep = jnp.where(cont, 1.0, 0.0).astype(jnp.float32)
        acc = tuple(a * keep for a in acc)
        cnt = cnt * keep

        def _row(r, acc):
            base = r * D
            return tuple(
                acc[j] + data_v[pl.ds(base + j * L, L)] for j in range(DSL)
            )

        acc = lax.fori_loop(0, CHUNK, _row, acc)
        cnt = cnt + jnp.float32(CHUNK)

        @pl.when(jnp.logical_not(uniform))
        def _():
            def _group(g, _):
                b16 = bid_v[pl.ds(c * CHUNK + g * L, L)]
                for r in range(L):
                    s_r = b16[r]
                    base = (g * L + r) * D
                    for j in range(DSL):
                        idx = s_r * D + j * L + lanes
                        plsc.addupdate_scatter(
                            acc_tab, [idx], data_v[pl.ds(base + j * L, L)]
                        )
                    plsc.addupdate_scatter(
                        cnt_tab,
                        [jnp.zeros((L,), jnp.int32) + s_r],
                        zf + 1.0,
                        mask=lane0,
                    )
                return 0

            lax.fori_loop(0, CHUNK // L, _group, 0)

        uf = jnp.where(uniform, 1.0, 0.0).astype(jnp.float32)
        acc = tuple(a * uf for a in acc)
        cnt = cnt * uf
        cur = jnp.where(uniform, first, last)
        return (cur, cnt, acc)

    cur0 = bid_v[pl.ds(0, L)][0]
    cur, cnt, acc = lax.fori_loop(
        0, NCHUNK, _chunk_body, (cur0, zero_cnt, zeros_acc)
    )
    _flush(cur, cnt, acc)

    pltpu.sync_copy(acc_tab, sums_hbm.at[wid])
    pltpu.sync_copy(cnt_tab, cnts_hbm.at[wid])


_sc_call = pl.kernel(
    _sc_body,
    out_type=(
        jax.ShapeDtypeStruct((NW, NSEG * D), jnp.float32),
        jax.ShapeDtypeStruct((NW, NSEG), jnp.float32),
    ),
    mesh=_mesh,
    compiler_params=pltpu.CompilerParams(needs_layout_passes=False),
    scratch_types=(
        pltpu.VMEM((CHUNK * D,), jnp.float32),
        pltpu.VMEM((ROWS_W,), jnp.int32),
        pltpu.VMEM((NSEG * D,), jnp.float32),
        pltpu.VMEM((NSEG,), jnp.float32),
    ),
)


def _combine_body(sums_ref, cnts_ref, out_ref):
    s = jnp.sum(sums_ref[...], axis=0).reshape(NSEG, D)
    c = jnp.sum(cnts_ref[...], axis=0).reshape(NSEG, 1)
    out_ref[...] = s / jnp.maximum(c, 1.0)


_combine = pl.pallas_call(
    _combine_body,
    out_shape=jax.ShapeDtypeStruct((NSEG, D), jnp.float32),
)


def kernel(data, batch_id, depth):
    sums, cnts = _sc_call(data.reshape(-1), batch_id)
    return _combine(sums, cnts)
